# baseline (speedup 1.0000x reference)
import functools

import jax
import jax.numpy as jnp
from jax import lax
from jax.experimental import pallas as pl
from jax.experimental.pallas import tpu as pltpu
from jax.experimental.pallas import tpu_sc as plsc

N = 10000
E = 320000
D = 128
G = 128
OUT = 10

NC = 2
NS = 16
NW = NC * NS
CHUNK = 80
PCH = 32
SLOW_CORE = 0
NCH_FAST = 256
NCH_SLOW = 0
TOTC = NS * (NCH_FAST + NCH_SLOW)
EPAD = TOTC * CHUNK
NPAD = 10240
ZCH = NPAD // NS // CHUNK


def _seg_sum_sc(h, src_t, dst_t):
  mesh = plsc.VectorSubcoreMesh(core_axis_name="c", subcore_axis_name="s")

  @functools.partial(
      pl.kernel,
      mesh=mesh,
      out_type=jax.ShapeDtypeStruct((NC, NPAD, D), jnp.float32),
      scratch_types=[
          pltpu.VMEM((2, PCH, CHUNK), jnp.int32),
          pltpu.VMEM((2, PCH, CHUNK), jnp.int32),
          pltpu.VMEM((CHUNK, D), jnp.float32),
          pltpu.VMEM((CHUNK, D), jnp.float32),
          pltpu.VMEM_SHARED((NPAD, D), jnp.float32),
          pltpu.SemaphoreType.DMA,
          pltpu.SemaphoreType.DMA,
          pltpu.SemaphoreType.DMA,
      ],
  )
  def k(h_hbm, src_hbm, dst_hbm, out_hbm, src_v, dst_v, rows0, rows1,
        acc_sh, sem0, sem1, semi):
    c = lax.axis_index("c")
    s = lax.axis_index("s")

    @pl.loop(0, CHUNK)
    def _(r):
      @pl.loop(0, D // 16)
      def _(k16):
        rows0.at[r, pl.ds(k16 * 16, 16)][...] = jnp.zeros((16,), jnp.float32)

    z0 = s * (NPAD // NS)
    for j in range(ZCH):
      pltpu.sync_copy(rows0, acc_sh.at[pl.ds(z0 + j * CHUNK, CHUNK)])

    plsc.subcore_barrier()

    is_fast = c != SLOW_CORE
    base = jnp.where(is_fast, s * NCH_FAST,
                     jnp.minimum(NS * NCH_FAST + s * NCH_SLOW, TOTC - PCH))
    nph = jnp.where(is_fast, NCH_FAST // PCH, NCH_SLOW // PCH)

    pltpu.sync_copy(src_hbm.at[pl.ds(base, PCH)], src_v.at[0])
    pltpu.sync_copy(dst_hbm.at[pl.ds(base, PCH)], dst_v.at[0])

    @pl.loop(0, nph)
    def _(q):
      qb = lax.rem(q, 2)
      nqb = lax.rem(q + 1, 2)

      @pl.when(q > 0)
      def _():
        pltpu.make_async_copy(
            src_hbm.at[pl.ds(base + q * PCH, PCH)], src_v.at[qb], semi
        ).wait()
        pltpu.make_async_copy(
            dst_hbm.at[pl.ds(base + q * PCH, PCH)], dst_v.at[qb], semi
        ).wait()

      @pl.when(q + 1 < nph)
      def _():
        pltpu.async_copy(
            src_hbm.at[pl.ds(base + (q + 1) * PCH, PCH)], src_v.at[nqb],
            semi)
        pltpu.async_copy(
            dst_hbm.at[pl.ds(base + (q + 1) * PCH, PCH)], dst_v.at[nqb],
            semi)

      sidx = src_v.at[qb]
      didx = dst_v.at[qb]
      pltpu.async_copy(h_hbm.at[sidx.at[0]], rows0, sem0)

      @pl.loop(0, PCH // 2)
      def _(i):
        j = i * 2
        pltpu.async_copy(h_hbm.at[sidx.at[j + 1]], rows1, sem1)
        pltpu.make_async_copy(h_hbm.at[sidx.at[j]], rows0, sem0).wait()
        pltpu.sync_copy(rows0, acc_sh.at[didx.at[j]], add=True)

        @pl.when(j + 2 < PCH)
        def _():
          pltpu.async_copy(h_hbm.at[sidx.at[j + 2]], rows0, sem0)

        pltpu.make_async_copy(h_hbm.at[sidx.at[j + 1]], rows1, sem1).wait()
        pltpu.sync_copy(rows1, acc_sh.at[didx.at[j + 1]], add=True)

    plsc.subcore_barrier()

    pltpu.sync_copy(acc_sh.at[pl.ds(z0, NPAD // NS)],
                    out_hbm.at[c].at[pl.ds(z0, NPAD // NS)])

  return k(h, src_t, dst_t)


_PREC = lax.Precision.HIGHEST


def _gin_ln_body(t_ref, p0_ref, p1_ref, w1_ref, b1_ref, w2_ref, b2_ref,
                 g_ref, bb_ref, o_ref):
  h = t_ref[...] + p0_ref[0] + p1_ref[0]
  a = jnp.maximum(
      jnp.dot(h, w1_ref[...], precision=_PREC,
              preferred_element_type=jnp.float32) + b1_ref[...], 0.0)
  hp = jnp.dot(a, w2_ref[...], precision=_PREC,
               preferred_element_type=jnp.float32) + b2_ref[...]
  r = jnp.maximum(hp, 0.0)
  mu = jnp.mean(r, axis=-1, keepdims=True)
  var = jnp.mean((r - mu) ** 2, axis=-1, keepdims=True)
  o_ref[...] = (r - mu) * lax.rsqrt(var + 1e-5) * g_ref[...] + bb_ref[...]


BLK = 400
NBLK = N // BLK


def _gin_ln_tc(t, parts, w1, b1, w2, b2, g, bb):
  return pl.pallas_call(
      _gin_ln_body,
      grid=(NBLK,),
      in_specs=[
          pl.BlockSpec((BLK, D), lambda i: (i, 0)),
          pl.BlockSpec((1, BLK, D), lambda i: (0, i, 0)),
          pl.BlockSpec((1, BLK, D), lambda i: (1, i, 0)),
          pl.BlockSpec((D, D), lambda i: (0, 0)),
          pl.BlockSpec((1, D), lambda i: (0, 0)),
          pl.BlockSpec((D, D), lambda i: (0, 0)),
          pl.BlockSpec((1, D), lambda i: (0, 0)),
          pl.BlockSpec((1, D), lambda i: (0, 0)),
          pl.BlockSpec((1, D), lambda i: (0, 0)),
      ],
      out_specs=pl.BlockSpec((BLK, D), lambda i: (i, 0)),
      out_shape=jax.ShapeDtypeStruct((N, D), jnp.float32),
  )(t, parts, parts, w1, b1, w2, b2, g, bb)


def _final_body(t_ref, p0_ref, p1_ref, w1_ref, b1_ref, w2_ref, b2_ref,
                batch_ref, wp1_ref, bp1_ref, wp2_ref, bp2_ref,
                emb_ref, logp_ref, pooled_acc, cnt_acc):
  i = pl.program_id(0)

  h = t_ref[...] + p0_ref[0] + p1_ref[0]
  a = jnp.maximum(
      jnp.dot(h, w1_ref[...], precision=_PREC,
              preferred_element_type=jnp.float32) + b1_ref[...], 0.0)
  hp = jnp.dot(a, w2_ref[...], precision=_PREC,
               preferred_element_type=jnp.float32) + b2_ref[...]
  emb_ref[...] = hp
  r = jnp.maximum(hp, 0.0)

  gid = lax.broadcasted_iota(jnp.int32, (BLK, G), 1)
  onehot = (batch_ref[...] == gid).astype(jnp.float32)
  dn = (((0,), (0,)), ((), ()))
  psum = lax.dot_general(onehot, r, dn, precision=_PREC,
                         preferred_element_type=jnp.float32)
  csum = lax.dot_general(onehot, jnp.ones((BLK, 1), jnp.float32), dn,
                         precision=_PREC, preferred_element_type=jnp.float32)

  @pl.when(i == 0)
  def _():
    pooled_acc[...] = jnp.zeros_like(pooled_acc)
    cnt_acc[...] = jnp.zeros_like(cnt_acc)

  pooled_acc[...] += psum
  cnt_acc[...] += csum

  @pl.when(i == NBLK - 1)
  def _():
    pooled = pooled_acc[...] / jnp.maximum(cnt_acc[...], 1.0)
    o1 = jnp.dot(pooled, wp1_ref[...], precision=_PREC,
                 preferred_element_type=jnp.float32) + bp1_ref[...]
    o2 = jnp.dot(o1, wp2_ref[...], precision=_PREC,
                 preferred_element_type=jnp.float32) + bp2_ref[...]
    m = jnp.max(o2, axis=1, keepdims=True)
    lse = jnp.log(jnp.sum(jnp.exp(o2 - m), axis=1, keepdims=True)) + m
    logp_ref[...] = o2 - lse


def _final_tc(t, parts, w1, b1, w2, b2, batch2d, wp1, bp1, wp2, bp2):
  return pl.pallas_call(
      _final_body,
      grid=(NBLK,),
      in_specs=[
          pl.BlockSpec((BLK, D), lambda i: (i, 0)),
          pl.BlockSpec((1, BLK, D), lambda i: (0, i, 0)),
          pl.BlockSpec((1, BLK, D), lambda i: (1, i, 0)),
          pl.BlockSpec((D, D), lambda i: (0, 0)),
          pl.BlockSpec((1, D), lambda i: (0, 0)),
          pl.BlockSpec((D, D), lambda i: (0, 0)),
          pl.BlockSpec((1, D), lambda i: (0, 0)),
          pl.BlockSpec((BLK, 1), lambda i: (i, 0)),
          pl.BlockSpec((D, D), lambda i: (0, 0)),
          pl.BlockSpec((1, D), lambda i: (0, 0)),
          pl.BlockSpec((D, OUT), lambda i: (0, 0)),
          pl.BlockSpec((1, OUT), lambda i: (0, 0)),
      ],
      out_specs=[
          pl.BlockSpec((BLK, D), lambda i: (i, 0)),
          pl.BlockSpec((G, OUT), lambda i: (0, 0)),
      ],
      out_shape=[
          jax.ShapeDtypeStruct((N, D), jnp.float32),
          jax.ShapeDtypeStruct((G, OUT), jnp.float32),
      ],
      scratch_shapes=[
          pltpu.VMEM((G, D), jnp.float32),
          pltpu.VMEM((G, 1), jnp.float32),
      ],
  )(t, parts, parts, w1, b1, w2, b2, batch2d, wp1, bp1, wp2, bp2)


def kernel(x, edge_index, batch,
           W1_0, b1_0, W2_0, b2_0,
           W1_1, b1_1, W2_1, b2_1,
           W1_2, b1_2, W2_2, b2_2,
           ln0_g, ln0_b, ln1_g, ln1_b,
           Wp1, bp1, Wp2, bp2):
  src = edge_index[0]
  dst = edge_index[1]
  pad = EPAD - E
  src_t = jnp.concatenate([src, jnp.zeros((pad,), jnp.int32)]).reshape(
      TOTC, CHUNK)
  dst_t = jnp.concatenate([dst, jnp.full((pad,), N, jnp.int32)]).reshape(
      TOTC, CHUNK)
  batch2d = batch.reshape(N, 1)
  row = lambda v: v.reshape(1, -1)

  parts0 = _seg_sum_sc(x, src_t, dst_t)
  t1 = _gin_ln_tc(x, parts0, W1_0, row(b1_0), W2_0, row(b2_0),
                  row(ln0_g), row(ln0_b))
  parts1 = _seg_sum_sc(t1, src_t, dst_t)
  t2 = _gin_ln_tc(t1, parts1, W1_1, row(b1_1), W2_1, row(b2_1),
                  row(ln1_g), row(ln1_b))
  parts2 = _seg_sum_sc(t2, src_t, dst_t)
  emb, logp = _final_tc(t2, parts2, W1_2, row(b1_2), W2_2, row(b2_2),
                        batch2d, Wp1, row(bp1), Wp2, row(bp2))
  return (emb, logp)


# speedup vs baseline: 3.4929x; 3.4929x over previous
import functools

import jax
import jax.numpy as jnp
from jax import lax
from jax.experimental import pallas as pl
from jax.experimental.pallas import tpu as pltpu
from jax.experimental.pallas import tpu_sc as plsc

N = 10000
E = 320000
D = 128
G = 128
OUT = 10

NC = 2
NS = 16
NW = NC * NS
CHUNK = 80
PCH = 32
SLOW_CORE = 0
NCH_FAST = 128
NCH_SLOW = 128
TOTC = NS * (NCH_FAST + NCH_SLOW)
EPAD = TOTC * CHUNK
NPAD = 10240
ZCH = NPAD // NS // CHUNK


def _seg_sum_sc(h, src_t, dst_t):
  mesh = plsc.VectorSubcoreMesh(core_axis_name="c", subcore_axis_name="s")

  @functools.partial(
      pl.kernel,
      mesh=mesh,
      out_type=jax.ShapeDtypeStruct((NC, NPAD, D), jnp.float32),
      scratch_types=[
          pltpu.VMEM((2, PCH, CHUNK), jnp.int32),
          pltpu.VMEM((2, PCH, CHUNK), jnp.int32),
          pltpu.VMEM((CHUNK, D), jnp.float32),
          pltpu.VMEM((CHUNK, D), jnp.float32),
          pltpu.VMEM_SHARED((NPAD, D), jnp.float32),
          pltpu.SemaphoreType.DMA,
          pltpu.SemaphoreType.DMA,
          pltpu.SemaphoreType.DMA,
      ],
  )
  def k(h_hbm, src_hbm, dst_hbm, out_hbm, src_v, dst_v, rows0, rows1,
        acc_sh, sem0, sem1, semi):
    c = lax.axis_index("c")
    s = lax.axis_index("s")

    @pl.loop(0, CHUNK)
    def _(r):
      @pl.loop(0, D // 16)
      def _(k16):
        rows0.at[r, pl.ds(k16 * 16, 16)][...] = jnp.zeros((16,), jnp.float32)

    z0 = s * (NPAD // NS)
    for j in range(ZCH):
      pltpu.sync_copy(rows0, acc_sh.at[pl.ds(z0 + j * CHUNK, CHUNK)])

    plsc.subcore_barrier()

    is_fast = c != SLOW_CORE
    base = jnp.where(is_fast, s * NCH_FAST,
                     jnp.minimum(NS * NCH_FAST + s * NCH_SLOW, TOTC - PCH))
    nph = jnp.where(is_fast, NCH_FAST // PCH, NCH_SLOW // PCH)

    pltpu.sync_copy(src_hbm.at[pl.ds(base, PCH)], src_v.at[0])
    pltpu.sync_copy(dst_hbm.at[pl.ds(base, PCH)], dst_v.at[0])

    @pl.loop(0, nph)
    def _(q):
      qb = lax.rem(q, 2)
      nqb = lax.rem(q + 1, 2)

      @pl.when(q > 0)
      def _():
        pltpu.make_async_copy(
            src_hbm.at[pl.ds(base + q * PCH, PCH)], src_v.at[qb], semi
        ).wait()
        pltpu.make_async_copy(
            dst_hbm.at[pl.ds(base + q * PCH, PCH)], dst_v.at[qb], semi
        ).wait()

      @pl.when(q + 1 < nph)
      def _():
        pltpu.async_copy(
            src_hbm.at[pl.ds(base + (q + 1) * PCH, PCH)], src_v.at[nqb],
            semi)
        pltpu.async_copy(
            dst_hbm.at[pl.ds(base + (q + 1) * PCH, PCH)], dst_v.at[nqb],
            semi)

      sidx = src_v.at[qb]
      didx = dst_v.at[qb]
      pltpu.async_copy(h_hbm.at[sidx.at[0]], rows0, sem0)

      @pl.loop(0, PCH // 2)
      def _(i):
        j = i * 2
        pltpu.async_copy(h_hbm.at[sidx.at[j + 1]], rows1, sem1)
        pltpu.make_async_copy(h_hbm.at[sidx.at[j]], rows0, sem0).wait()
        pltpu.sync_copy(rows0, acc_sh.at[didx.at[j]], add=True)

        @pl.when(j + 2 < PCH)
        def _():
          pltpu.async_copy(h_hbm.at[sidx.at[j + 2]], rows0, sem0)

        pltpu.make_async_copy(h_hbm.at[sidx.at[j + 1]], rows1, sem1).wait()
        pltpu.sync_copy(rows1, acc_sh.at[didx.at[j + 1]], add=True)

    plsc.subcore_barrier()

    pltpu.sync_copy(acc_sh.at[pl.ds(z0, NPAD // NS)],
                    out_hbm.at[c].at[pl.ds(z0, NPAD // NS)])

  return k(h, src_t, dst_t)


_PREC = lax.Precision.HIGHEST


def _gin_ln_body(t_ref, p0_ref, p1_ref, w1_ref, b1_ref, w2_ref, b2_ref,
                 g_ref, bb_ref, o_ref):
  h = t_ref[...] + p0_ref[0] + p1_ref[0]
  a = jnp.maximum(
      jnp.dot(h, w1_ref[...], precision=_PREC,
              preferred_element_type=jnp.float32) + b1_ref[...], 0.0)
  hp = jnp.dot(a, w2_ref[...], precision=_PREC,
               preferred_element_type=jnp.float32) + b2_ref[...]
  r = jnp.maximum(hp, 0.0)
  mu = jnp.mean(r, axis=-1, keepdims=True)
  var = jnp.mean((r - mu) ** 2, axis=-1, keepdims=True)
  o_ref[...] = (r - mu) * lax.rsqrt(var + 1e-5) * g_ref[...] + bb_ref[...]


BLK = 400
NBLK = N // BLK


def _gin_ln_tc(t, parts, w1, b1, w2, b2, g, bb):
  return pl.pallas_call(
      _gin_ln_body,
      grid=(NBLK,),
      in_specs=[
          pl.BlockSpec((BLK, D), lambda i: (i, 0)),
          pl.BlockSpec((1, BLK, D), lambda i: (0, i, 0)),
          pl.BlockSpec((1, BLK, D), lambda i: (1, i, 0)),
          pl.BlockSpec((D, D), lambda i: (0, 0)),
          pl.BlockSpec((1, D), lambda i: (0, 0)),
          pl.BlockSpec((D, D), lambda i: (0, 0)),
          pl.BlockSpec((1, D), lambda i: (0, 0)),
          pl.BlockSpec((1, D), lambda i: (0, 0)),
          pl.BlockSpec((1, D), lambda i: (0, 0)),
      ],
      out_specs=pl.BlockSpec((BLK, D), lambda i: (i, 0)),
      out_shape=jax.ShapeDtypeStruct((N, D), jnp.float32),
  )(t, parts, parts, w1, b1, w2, b2, g, bb)


def _final_body(t_ref, p0_ref, p1_ref, w1_ref, b1_ref, w2_ref, b2_ref,
                batch_ref, wp1_ref, bp1_ref, wp2_ref, bp2_ref,
                emb_ref, logp_ref, pooled_acc, cnt_acc):
  i = pl.program_id(0)

  h = t_ref[...] + p0_ref[0] + p1_ref[0]
  a = jnp.maximum(
      jnp.dot(h, w1_ref[...], precision=_PREC,
              preferred_element_type=jnp.float32) + b1_ref[...], 0.0)
  hp = jnp.dot(a, w2_ref[...], precision=_PREC,
               preferred_element_type=jnp.float32) + b2_ref[...]
  emb_ref[...] = hp
  r = jnp.maximum(hp, 0.0)

  gid = lax.broadcasted_iota(jnp.int32, (BLK, G), 1)
  onehot = (batch_ref[...] == gid).astype(jnp.float32)
  dn = (((0,), (0,)), ((), ()))
  psum = lax.dot_general(onehot, r, dn, precision=_PREC,
                         preferred_element_type=jnp.float32)
  csum = lax.dot_general(onehot, jnp.ones((BLK, 1), jnp.float32), dn,
                         precision=_PREC, preferred_element_type=jnp.float32)

  @pl.when(i == 0)
  def _():
    pooled_acc[...] = jnp.zeros_like(pooled_acc)
    cnt_acc[...] = jnp.zeros_like(cnt_acc)

  pooled_acc[...] += psum
  cnt_acc[...] += csum

  @pl.when(i == NBLK - 1)
  def _():
    pooled = pooled_acc[...] / jnp.maximum(cnt_acc[...], 1.0)
    o1 = jnp.dot(pooled, wp1_ref[...], precision=_PREC,
                 preferred_element_type=jnp.float32) + bp1_ref[...]
    o2 = jnp.dot(o1, wp2_ref[...], precision=_PREC,
                 preferred_element_type=jnp.float32) + bp2_ref[...]
    m = jnp.max(o2, axis=1, keepdims=True)
    lse = jnp.log(jnp.sum(jnp.exp(o2 - m), axis=1, keepdims=True)) + m
    logp_ref[...] = o2 - lse


def _final_tc(t, parts, w1, b1, w2, b2, batch2d, wp1, bp1, wp2, bp2):
  return pl.pallas_call(
      _final_body,
      grid=(NBLK,),
      in_specs=[
          pl.BlockSpec((BLK, D), lambda i: (i, 0)),
          pl.BlockSpec((1, BLK, D), lambda i: (0, i, 0)),
          pl.BlockSpec((1, BLK, D), lambda i: (1, i, 0)),
          pl.BlockSpec((D, D), lambda i: (0, 0)),
          pl.BlockSpec((1, D), lambda i: (0, 0)),
          pl.BlockSpec((D, D), lambda i: (0, 0)),
          pl.BlockSpec((1, D), lambda i: (0, 0)),
          pl.BlockSpec((BLK, 1), lambda i: (i, 0)),
          pl.BlockSpec((D, D), lambda i: (0, 0)),
          pl.BlockSpec((1, D), lambda i: (0, 0)),
          pl.BlockSpec((D, OUT), lambda i: (0, 0)),
          pl.BlockSpec((1, OUT), lambda i: (0, 0)),
      ],
      out_specs=[
          pl.BlockSpec((BLK, D), lambda i: (i, 0)),
          pl.BlockSpec((G, OUT), lambda i: (0, 0)),
      ],
      out_shape=[
          jax.ShapeDtypeStruct((N, D), jnp.float32),
          jax.ShapeDtypeStruct((G, OUT), jnp.float32),
      ],
      scratch_shapes=[
          pltpu.VMEM((G, D), jnp.float32),
          pltpu.VMEM((G, 1), jnp.float32),
      ],
  )(t, parts, parts, w1, b1, w2, b2, batch2d, wp1, bp1, wp2, bp2)


def kernel(x, edge_index, batch,
           W1_0, b1_0, W2_0, b2_0,
           W1_1, b1_1, W2_1, b2_1,
           W1_2, b1_2, W2_2, b2_2,
           ln0_g, ln0_b, ln1_g, ln1_b,
           Wp1, bp1, Wp2, bp2):
  src = edge_index[0]
  dst = edge_index[1]
  pad = EPAD - E
  ar = jnp.arange(pad, dtype=jnp.int32)
  pad_src = (ar * 997) % N
  pad_dst = N + ar % (NPAD - N)
  src_t = jnp.concatenate([src, pad_src]).reshape(TOTC, CHUNK)
  dst_t = jnp.concatenate([dst, pad_dst]).reshape(TOTC, CHUNK)
  batch2d = batch.reshape(N, 1)
  row = lambda v: v.reshape(1, -1)

  parts0 = _seg_sum_sc(x, src_t, dst_t)
  t1 = _gin_ln_tc(x, parts0, W1_0, row(b1_0), W2_0, row(b2_0),
                  row(ln0_g), row(ln0_b))
  parts1 = _seg_sum_sc(t1, src_t, dst_t)
  t2 = _gin_ln_tc(t1, parts1, W1_1, row(b1_1), W2_1, row(b2_1),
                  row(ln1_g), row(ln1_b))
  parts2 = _seg_sum_sc(t2, src_t, dst_t)
  emb, logp = _final_tc(t2, parts2, W1_2, row(b1_2), W2_2, row(b2_2),
                        batch2d, Wp1, row(bp1), Wp2, row(bp2))
  return (emb, logp)


# speedup vs baseline: 3.8313x; 1.0969x over previous
import functools

import jax
import jax.numpy as jnp
from jax import lax
from jax.experimental import pallas as pl
from jax.experimental.pallas import tpu as pltpu
from jax.experimental.pallas import tpu_sc as plsc

N = 10000
E = 320000
D = 128
G = 128
OUT = 10

NC = 2
NS = 16
NW = NC * NS
CHUNK = 64
PCH = 16
NCH = 160
TOTC = NW * NCH
EPAD = TOTC * CHUNK
NPAD = 10112
STRIPE = NPAD // NS


def _seg_sum_sc(h, src_t, dst_t):
  mesh = plsc.VectorSubcoreMesh(core_axis_name="c", subcore_axis_name="s")

  @functools.partial(
      pl.kernel,
      mesh=mesh,
      out_type=jax.ShapeDtypeStruct((NC, NPAD, D), jnp.float32),
      scratch_types=[
          pltpu.VMEM((2, PCH, CHUNK), jnp.int32),
          pltpu.VMEM((2, PCH, CHUNK), jnp.int32),
          pltpu.VMEM((CHUNK, D), jnp.float32),
          pltpu.VMEM((CHUNK, D), jnp.float32),
          pltpu.VMEM((CHUNK, D), jnp.float32),
          pltpu.VMEM((CHUNK, D), jnp.float32),
          pltpu.VMEM_SHARED((NPAD, D), jnp.float32),
          pltpu.SemaphoreType.DMA,
          pltpu.SemaphoreType.DMA,
          pltpu.SemaphoreType.DMA,
          pltpu.SemaphoreType.DMA,
          pltpu.SemaphoreType.DMA,
      ],
  )
  def k(h_hbm, src_hbm, dst_hbm, out_hbm, src_v, dst_v, rows0, rows1,
        rows2, rows3, acc_sh, sem0, sem1, sem2, sem3, semi):
    c = lax.axis_index("c")
    s = lax.axis_index("s")
    rows = (rows0, rows1, rows2, rows3)
    sems = (sem0, sem1, sem2, sem3)

    @pl.loop(0, CHUNK)
    def _(r):
      @pl.loop(0, D // 16)
      def _(k16):
        rows0.at[r, pl.ds(k16 * 16, 16)][...] = jnp.zeros((16,), jnp.float32)

    z0 = s * STRIPE
    for j in range(STRIPE // CHUNK):
      pltpu.sync_copy(rows0, acc_sh.at[pl.ds(z0 + j * CHUNK, CHUNK)])
    rem_rows = STRIPE % CHUNK
    if rem_rows:
      pltpu.sync_copy(rows0.at[pl.ds(0, rem_rows)],
                      acc_sh.at[pl.ds(z0 + STRIPE - rem_rows, rem_rows)])

    plsc.subcore_barrier()

    wid = s * NC + c
    base = wid * NCH

    def gth(sidx, j, r):
      pltpu.async_copy(h_hbm.at[sidx.at[j]], rows[r], sems[r])

    def gwait(sidx, j, r):
      pltpu.make_async_copy(h_hbm.at[sidx.at[j]], rows[r], sems[r]).wait()

    def sadd(didx, j, r):
      pltpu.sync_copy(rows[r], acc_sh.at[didx.at[j]], add=True)

    pltpu.sync_copy(src_hbm.at[pl.ds(base, PCH)], src_v.at[0])
    pltpu.sync_copy(dst_hbm.at[pl.ds(base, PCH)], dst_v.at[0])

    @pl.loop(0, NCH // PCH)
    def _(q):
      qb = lax.rem(q, 2)
      nqb = lax.rem(q + 1, 2)

      @pl.when(q > 0)
      def _():
        pltpu.make_async_copy(
            src_hbm.at[pl.ds(base + q * PCH, PCH)], src_v.at[qb], semi
        ).wait()
        pltpu.make_async_copy(
            dst_hbm.at[pl.ds(base + q * PCH, PCH)], dst_v.at[qb], semi
        ).wait()

      @pl.when(q + 1 < NCH // PCH)
      def _():
        pltpu.async_copy(
            src_hbm.at[pl.ds(base + (q + 1) * PCH, PCH)], src_v.at[nqb],
            semi)
        pltpu.async_copy(
            dst_hbm.at[pl.ds(base + (q + 1) * PCH, PCH)], dst_v.at[nqb],
            semi)

      sidx = src_v.at[qb]
      didx = dst_v.at[qb]
      gth(sidx, 0, 0)
      gth(sidx, 1, 1)
      gth(sidx, 2, 2)

      @pl.loop(0, PCH // 4)
      def _(i):
        j = i * 4
        gth(sidx, j + 3, 3)
        gwait(sidx, j, 0)
        sadd(didx, j, 0)

        @pl.when(j + 4 < PCH)
        def _():
          gth(sidx, j + 4, 0)

        gwait(sidx, j + 1, 1)
        sadd(didx, j + 1, 1)

        @pl.when(j + 5 < PCH)
        def _():
          gth(sidx, j + 5, 1)

        gwait(sidx, j + 2, 2)
        sadd(didx, j + 2, 2)

        @pl.when(j + 6 < PCH)
        def _():
          gth(sidx, j + 6, 2)

        gwait(sidx, j + 3, 3)
        sadd(didx, j + 3, 3)

    plsc.subcore_barrier()

    pltpu.sync_copy(acc_sh.at[pl.ds(z0, STRIPE)],
                    out_hbm.at[c].at[pl.ds(z0, STRIPE)])

  return k(h, src_t, dst_t)


_PREC = lax.Precision.HIGHEST


def _gin_ln_body(t_ref, p0_ref, p1_ref, w1_ref, b1_ref, w2_ref, b2_ref,
                 g_ref, bb_ref, o_ref):
  h = t_ref[...] + p0_ref[0] + p1_ref[0]
  a = jnp.maximum(
      jnp.dot(h, w1_ref[...], precision=_PREC,
              preferred_element_type=jnp.float32) + b1_ref[...], 0.0)
  hp = jnp.dot(a, w2_ref[...], precision=_PREC,
               preferred_element_type=jnp.float32) + b2_ref[...]
  r = jnp.maximum(hp, 0.0)
  mu = jnp.mean(r, axis=-1, keepdims=True)
  var = jnp.mean((r - mu) ** 2, axis=-1, keepdims=True)
  o_ref[...] = (r - mu) * lax.rsqrt(var + 1e-5) * g_ref[...] + bb_ref[...]


BLK = 400
NBLK = N // BLK


def _gin_ln_tc(t, parts, w1, b1, w2, b2, g, bb):
  return pl.pallas_call(
      _gin_ln_body,
      grid=(NBLK,),
      in_specs=[
          pl.BlockSpec((BLK, D), lambda i: (i, 0)),
          pl.BlockSpec((1, BLK, D), lambda i: (0, i, 0)),
          pl.BlockSpec((1, BLK, D), lambda i: (1, i, 0)),
          pl.BlockSpec((D, D), lambda i: (0, 0)),
          pl.BlockSpec((1, D), lambda i: (0, 0)),
          pl.BlockSpec((D, D), lambda i: (0, 0)),
          pl.BlockSpec((1, D), lambda i: (0, 0)),
          pl.BlockSpec((1, D), lambda i: (0, 0)),
          pl.BlockSpec((1, D), lambda i: (0, 0)),
      ],
      out_specs=pl.BlockSpec((BLK, D), lambda i: (i, 0)),
      out_shape=jax.ShapeDtypeStruct((N, D), jnp.float32),
  )(t, parts, parts, w1, b1, w2, b2, g, bb)


def _final_body(t_ref, p0_ref, p1_ref, w1_ref, b1_ref, w2_ref, b2_ref,
                batch_ref, wp1_ref, bp1_ref, wp2_ref, bp2_ref,
                emb_ref, logp_ref, pooled_acc, cnt_acc):
  i = pl.program_id(0)

  h = t_ref[...] + p0_ref[0] + p1_ref[0]
  a = jnp.maximum(
      jnp.dot(h, w1_ref[...], precision=_PREC,
              preferred_element_type=jnp.float32) + b1_ref[...], 0.0)
  hp = jnp.dot(a, w2_ref[...], precision=_PREC,
               preferred_element_type=jnp.float32) + b2_ref[...]
  emb_ref[...] = hp
  r = jnp.maximum(hp, 0.0)

  gid = lax.broadcasted_iota(jnp.int32, (BLK, G), 1)
  onehot = (batch_ref[...] == gid).astype(jnp.float32)
  dn = (((0,), (0,)), ((), ()))
  psum = lax.dot_general(onehot, r, dn, precision=_PREC,
                         preferred_element_type=jnp.float32)
  csum = lax.dot_general(onehot, jnp.ones((BLK, 1), jnp.float32), dn,
                         precision=_PREC, preferred_element_type=jnp.float32)

  @pl.when(i == 0)
  def _():
    pooled_acc[...] = jnp.zeros_like(pooled_acc)
    cnt_acc[...] = jnp.zeros_like(cnt_acc)

  pooled_acc[...] += psum
  cnt_acc[...] += csum

  @pl.when(i == NBLK - 1)
  def _():
    pooled = pooled_acc[...] / jnp.maximum(cnt_acc[...], 1.0)
    o1 = jnp.dot(pooled, wp1_ref[...], precision=_PREC,
                 preferred_element_type=jnp.float32) + bp1_ref[...]
    o2 = jnp.dot(o1, wp2_ref[...], precision=_PREC,
                 preferred_element_type=jnp.float32) + bp2_ref[...]
    m = jnp.max(o2, axis=1, keepdims=True)
    lse = jnp.log(jnp.sum(jnp.exp(o2 - m), axis=1, keepdims=True)) + m
    logp_ref[...] = o2 - lse


def _final_tc(t, parts, w1, b1, w2, b2, batch2d, wp1, bp1, wp2, bp2):
  return pl.pallas_call(
      _final_body,
      grid=(NBLK,),
      in_specs=[
          pl.BlockSpec((BLK, D), lambda i: (i, 0)),
          pl.BlockSpec((1, BLK, D), lambda i: (0, i, 0)),
          pl.BlockSpec((1, BLK, D), lambda i: (1, i, 0)),
          pl.BlockSpec((D, D), lambda i: (0, 0)),
          pl.BlockSpec((1, D), lambda i: (0, 0)),
          pl.BlockSpec((D, D), lambda i: (0, 0)),
          pl.BlockSpec((1, D), lambda i: (0, 0)),
          pl.BlockSpec((BLK, 1), lambda i: (i, 0)),
          pl.BlockSpec((D, D), lambda i: (0, 0)),
          pl.BlockSpec((1, D), lambda i: (0, 0)),
          pl.BlockSpec((D, OUT), lambda i: (0, 0)),
          pl.BlockSpec((1, OUT), lambda i: (0, 0)),
      ],
      out_specs=[
          pl.BlockSpec((BLK, D), lambda i: (i, 0)),
          pl.BlockSpec((G, OUT), lambda i: (0, 0)),
      ],
      out_shape=[
          jax.ShapeDtypeStruct((N, D), jnp.float32),
          jax.ShapeDtypeStruct((G, OUT), jnp.float32),
      ],
      scratch_shapes=[
          pltpu.VMEM((G, D), jnp.float32),
          pltpu.VMEM((G, 1), jnp.float32),
      ],
  )(t, parts, parts, w1, b1, w2, b2, batch2d, wp1, bp1, wp2, bp2)


def kernel(x, edge_index, batch,
           W1_0, b1_0, W2_0, b2_0,
           W1_1, b1_1, W2_1, b2_1,
           W1_2, b1_2, W2_2, b2_2,
           ln0_g, ln0_b, ln1_g, ln1_b,
           Wp1, bp1, Wp2, bp2):
  src = edge_index[0]
  dst = edge_index[1]
  pad = EPAD - E
  ar = jnp.arange(pad, dtype=jnp.int32)
  pad_src = (ar * 997) % N
  pad_dst = N + ar % (NPAD - N)
  src_t = jnp.concatenate([src, pad_src]).reshape(TOTC, CHUNK)
  dst_t = jnp.concatenate([dst, pad_dst]).reshape(TOTC, CHUNK)
  batch2d = batch.reshape(N, 1)
  row = lambda v: v.reshape(1, -1)

  parts0 = _seg_sum_sc(x, src_t, dst_t)
  t1 = _gin_ln_tc(x, parts0, W1_0, row(b1_0), W2_0, row(b2_0),
                  row(ln0_g), row(ln0_b))
  parts1 = _seg_sum_sc(t1, src_t, dst_t)
  t2 = _gin_ln_tc(t1, parts1, W1_1, row(b1_1), W2_1, row(b2_1),
                  row(ln1_g), row(ln1_b))
  parts2 = _seg_sum_sc(t2, src_t, dst_t)
  emb, logp = _final_tc(t2, parts2, W1_2, row(b1_2), W2_2, row(b2_2),
                        batch2d, Wp1, row(bp1), Wp2, row(bp2))
  return (emb, logp)
